# baseline (speedup 1.0000x reference)
import jax
import jax.numpy as jnp
from jax.experimental import pallas as pl
from jax.experimental.pallas import tpu as pltpu


def _gate_body(x_ref, w1_ref, w2_ref, b1_ref, b2_ref, o_ref):
    Bb, C, HW = x_ref.shape
    inv_hw = 1.0 / HW

    ones_m = jnp.full((HW, Bb), inv_hw, dtype=jnp.float32)
    lane_id = jax.lax.broadcasted_iota(jnp.int32, (C, Bb), 1)

    avg_mat = jnp.zeros((C, Bb), dtype=jnp.float32)
    max_mat = jnp.zeros((C, Bb), dtype=jnp.float32)
    for b in range(Bb):
        xb = x_ref[b]
        avg_rep = jax.lax.dot_general(
            xb, ones_m, (((1,), (0,)), ((), ())),
            preferred_element_type=jnp.float32)
        mx_col = jnp.max(xb, axis=-1, keepdims=True)
        sel = lane_id == b
        avg_mat = jnp.where(sel, avg_rep, avg_mat)
        max_mat = jnp.where(sel, mx_col, max_mat)

    dn = (((1,), (0,)), ((), ()))
    h_a = jax.lax.dot_general(w1_ref[...], avg_mat, dn,
                              preferred_element_type=jnp.float32)
    h_m = jax.lax.dot_general(w1_ref[...], max_mat, dn,
                              preferred_element_type=jnp.float32)
    b1 = b1_ref[...]
    h_sum = jnp.maximum(h_a + b1, 0.0) + jnp.maximum(h_m + b1, 0.0)
    att = jax.lax.dot_general(w2_ref[...], h_sum, dn,
                              preferred_element_type=jnp.float32)
    scale = jax.nn.sigmoid(att + b2_ref[...])

    row_id = jax.lax.broadcasted_iota(jnp.int32, (Bb, HW), 0)
    for b in range(Bb):
        g = (row_id == b).astype(jnp.float32)
        sc_full = jax.lax.dot_general(
            scale, g, (((1,), (0,)), ((), ())),
            preferred_element_type=jnp.float32)
        o_ref[b] = x_ref[b] * sc_full


def _pick_bb(batch, per_batch_bytes, target_bytes=2 << 20):
    bb = max(1, min(batch, target_bytes // max(per_batch_bytes, 1)))
    while batch % bb != 0:
        bb -= 1
    return bb


def kernel(x, w1, b1_row, w2, b2_row):
    B, C, H, W = x.shape
    HW = H * W
    hid = w1.shape[0]

    per_batch_bytes = C * HW * x.dtype.itemsize
    Bb = _pick_bb(B, per_batch_bytes)
    steps = B // Bb

    x2 = x.reshape(B, C, HW)
    b1_mat = jnp.broadcast_to(b1_row.reshape(hid, 1), (hid, Bb))
    b2_mat = jnp.broadcast_to(2.0 * b2_row.reshape(C, 1), (C, Bb))

    feat_spec = pl.BlockSpec((Bb, C, HW), lambda i: (i, 0, 0))
    block_bytes = Bb * per_batch_bytes
    vmem_limit = int(min(56 << 20, 4 * block_bytes + (8 << 20)))

    out = pl.pallas_call(
        _gate_body,
        out_shape=jax.ShapeDtypeStruct((B, C, HW), x.dtype),
        grid=(steps,),
        in_specs=[
            feat_spec,
            pl.BlockSpec((hid, C), lambda i: (0, 0)),
            pl.BlockSpec((C, hid), lambda i: (0, 0)),
            pl.BlockSpec((hid, Bb), lambda i: (0, 0)),
            pl.BlockSpec((C, Bb), lambda i: (0, 0)),
        ],
        out_specs=feat_spec,
        compiler_params=pltpu.CompilerParams(
            dimension_semantics=("parallel",),
            vmem_limit_bytes=vmem_limit,
        ),
    )(x2, w1, w2, b1_mat, b2_mat)

    return out.reshape(B, C, H, W)


# speedup vs baseline: 1.0991x; 1.0991x over previous
import jax
import jax.numpy as jnp
from jax.experimental import pallas as pl
from jax.experimental.pallas import tpu as pltpu


def _gate_body(x_ref, w1_ref, w2_ref, b1_ref, b2_ref, o_ref):
    Bb, C, HW = x_ref.shape
    inv_hw = 1.0 / HW

    ones_m = jnp.full((HW, Bb), inv_hw, dtype=jnp.float32)
    lane_id = jax.lax.broadcasted_iota(jnp.int32, (C, Bb), 1)

    avg_mat = jnp.zeros((C, Bb), dtype=jnp.float32)
    max_mat = jnp.zeros((C, Bb), dtype=jnp.float32)
    for b in range(Bb):
        xb = x_ref[b]
        avg_rep = jax.lax.dot_general(
            xb, ones_m, (((1,), (0,)), ((), ())),
            preferred_element_type=jnp.float32)
        mx_col = jnp.max(xb, axis=-1, keepdims=True)
        sel = lane_id == b
        avg_mat = jnp.where(sel, avg_rep, avg_mat)
        max_mat = jnp.where(sel, mx_col, max_mat)

    dn = (((1,), (0,)), ((), ()))
    h_a = jax.lax.dot_general(w1_ref[...], avg_mat, dn,
                              preferred_element_type=jnp.float32)
    h_m = jax.lax.dot_general(w1_ref[...], max_mat, dn,
                              preferred_element_type=jnp.float32)
    b1 = b1_ref[...]
    h_sum = jnp.maximum(h_a + b1, 0.0) + jnp.maximum(h_m + b1, 0.0)
    att = jax.lax.dot_general(w2_ref[...], h_sum, dn,
                              preferred_element_type=jnp.float32)
    scale = jax.nn.sigmoid(att + b2_ref[...])

    row_id = jax.lax.broadcasted_iota(jnp.int32, (Bb, HW), 0)
    for b in range(Bb):
        g = (row_id == b).astype(jnp.float32)
        sc_full = jax.lax.dot_general(
            scale, g, (((1,), (0,)), ((), ())),
            preferred_element_type=jnp.float32)
        o_ref[b] = x_ref[b] * sc_full


def _pick_bb(batch, per_batch_bytes, target_bytes=4 << 20):
    bb = max(1, min(batch, target_bytes // max(per_batch_bytes, 1)))
    while batch % bb != 0:
        bb -= 1
    return bb


def kernel(x, w1, b1_row, w2, b2_row):
    B, C, H, W = x.shape
    HW = H * W
    hid = w1.shape[0]

    per_batch_bytes = C * HW * x.dtype.itemsize
    Bb = _pick_bb(B, per_batch_bytes)
    steps = B // Bb

    x2 = x.reshape(B, C, HW)
    b1_mat = jnp.broadcast_to(b1_row.reshape(hid, 1), (hid, Bb))
    b2_mat = jnp.broadcast_to(2.0 * b2_row.reshape(C, 1), (C, Bb))

    feat_spec = pl.BlockSpec((Bb, C, HW), lambda i: (i, 0, 0))
    block_bytes = Bb * per_batch_bytes
    vmem_limit = int(min(56 << 20, 4 * block_bytes + (8 << 20)))

    out = pl.pallas_call(
        _gate_body,
        out_shape=jax.ShapeDtypeStruct((B, C, HW), x.dtype),
        grid=(steps,),
        in_specs=[
            feat_spec,
            pl.BlockSpec((hid, C), lambda i: (0, 0)),
            pl.BlockSpec((C, hid), lambda i: (0, 0)),
            pl.BlockSpec((hid, Bb), lambda i: (0, 0)),
            pl.BlockSpec((C, Bb), lambda i: (0, 0)),
        ],
        out_specs=feat_spec,
        compiler_params=pltpu.CompilerParams(
            dimension_semantics=("parallel",),
            vmem_limit_bytes=vmem_limit,
        ),
    )(x2, w1, w2, b1_mat, b2_mat)

    return out.reshape(B, C, H, W)


# speedup vs baseline: 1.1053x; 1.0056x over previous
import jax
import jax.numpy as jnp
from jax.experimental import pallas as pl
from jax.experimental.pallas import tpu as pltpu


def _gate_body(x_ref, w1_ref, w2_ref, b1_ref, b2_ref, o_ref):
    Bb, C, HW = x_ref.shape
    inv_hw = 1.0 / HW

    ones_m = jnp.full((HW, Bb), inv_hw, dtype=jnp.float32)
    lane_id = jax.lax.broadcasted_iota(jnp.int32, (C, Bb), 1)

    avg_mat = jnp.zeros((C, Bb), dtype=jnp.float32)
    max_mat = jnp.zeros((C, Bb), dtype=jnp.float32)
    for b in range(Bb):
        xb = x_ref[b]
        avg_rep = jax.lax.dot_general(
            xb, ones_m, (((1,), (0,)), ((), ())),
            preferred_element_type=jnp.float32)
        mx_col = jnp.max(xb, axis=-1, keepdims=True)
        sel = lane_id == b
        avg_mat = jnp.where(sel, avg_rep, avg_mat)
        max_mat = jnp.where(sel, mx_col, max_mat)

    dn = (((1,), (0,)), ((), ()))
    h_a = jax.lax.dot_general(w1_ref[...], avg_mat, dn,
                              preferred_element_type=jnp.float32)
    h_m = jax.lax.dot_general(w1_ref[...], max_mat, dn,
                              preferred_element_type=jnp.float32)
    b1 = b1_ref[...]
    h_sum = jnp.maximum(h_a + b1, 0.0) + jnp.maximum(h_m + b1, 0.0)
    att = jax.lax.dot_general(w2_ref[...], h_sum, dn,
                              preferred_element_type=jnp.float32)
    scale = jax.nn.sigmoid(att + b2_ref[...])

    row_id = jax.lax.broadcasted_iota(jnp.int32, (Bb, HW), 0)
    for b in range(Bb):
        g = (row_id == b).astype(jnp.float32)
        sc_full = jax.lax.dot_general(
            scale, g, (((1,), (0,)), ((), ())),
            preferred_element_type=jnp.float32)
        o_ref[b] = x_ref[b] * sc_full


def _pick_bb(batch, per_batch_bytes, target_bytes=8 << 20):
    bb = max(1, min(batch, target_bytes // max(per_batch_bytes, 1)))
    while batch % bb != 0:
        bb -= 1
    return bb


def kernel(x, w1, b1_row, w2, b2_row):
    B, C, H, W = x.shape
    HW = H * W
    hid = w1.shape[0]

    per_batch_bytes = C * HW * x.dtype.itemsize
    Bb = _pick_bb(B, per_batch_bytes)
    steps = B // Bb

    x2 = x.reshape(B, C, HW)
    b1_mat = jnp.broadcast_to(b1_row.reshape(hid, 1), (hid, Bb))
    b2_mat = jnp.broadcast_to(2.0 * b2_row.reshape(C, 1), (C, Bb))

    feat_spec = pl.BlockSpec((Bb, C, HW), lambda i: (i, 0, 0))
    block_bytes = Bb * per_batch_bytes
    vmem_limit = int(min(56 << 20, 4 * block_bytes + (8 << 20)))

    out = pl.pallas_call(
        _gate_body,
        out_shape=jax.ShapeDtypeStruct((B, C, HW), x.dtype),
        grid=(steps,),
        in_specs=[
            feat_spec,
            pl.BlockSpec((hid, C), lambda i: (0, 0)),
            pl.BlockSpec((C, hid), lambda i: (0, 0)),
            pl.BlockSpec((hid, Bb), lambda i: (0, 0)),
            pl.BlockSpec((C, Bb), lambda i: (0, 0)),
        ],
        out_specs=feat_spec,
        compiler_params=pltpu.CompilerParams(
            dimension_semantics=("parallel",),
            vmem_limit_bytes=vmem_limit,
        ),
    )(x2, w1, w2, b1_mat, b2_mat)

    return out.reshape(B, C, H, W)
